# baseline (speedup 1.0000x reference)
import functools

import jax
import jax.numpy as jnp
from jax import lax
from jax.experimental import pallas as pl
from jax.experimental.pallas import tpu as pltpu
from jax.experimental.pallas import tpu_sc as plsc


def _rep(shape):
    return pl.BlockSpec(shape, lambda *_: (0,) * len(shape))


def _row(bm, cols):
    return pl.BlockSpec((bm, cols), lambda i: (i, 0))


def _dot(a, b):
    return jnp.dot(a, b, preferred_element_type=jnp.float32)


def _bdot(a, b):
    return jnp.dot(a.astype(jnp.bfloat16), b.astype(jnp.bfloat16),
                   preferred_element_type=jnp.float32)


def _k2_body(x_ref, g_ref, w1_ref, wb1_ref, b1_ref, w2_ref, b2_ref, wb_ref,
             o_ref, u_ref):
    h = jax.nn.relu(_dot(x_ref[...], w1_ref[...]) + _dot(g_ref[...], wb1_ref[...])
                    + b1_ref[...])
    o = jax.nn.relu(_dot(h, w2_ref[...]) + b2_ref[...])
    o_ref[...] = o
    u = _dot(o, wb_ref[...])
    half = u.shape[0] // 2
    u_ref[...] = jnp.concatenate([u[:half], u[half:]], axis=1)


def _item_mlp(x, g, g_off, w1, wb1, b1, w2, b2, wb, bm=2000):
    n = x.shape[0]
    ob = g_off // bm
    return pl.pallas_call(
        _k2_body,
        grid=(n // bm,),
        in_specs=[_row(bm, 16),
                  pl.BlockSpec((bm, 128), lambda i: (i + ob, 0)),
                  _rep((16, 64)), _rep((128, 64)), _rep((1, 64)),
                  _rep((64, 64)), _rep((1, 64)), _rep((64, 64))],
        out_specs=[_row(bm, 64), _row(bm // 2, 128)],
        out_shape=[jax.ShapeDtypeStruct((n, 64), jnp.float32),
                   jax.ShapeDtypeStruct((n // 2, 128), jnp.float32)],
        compiler_params=pltpu.CompilerParams(dimension_semantics=("parallel",)),
    )(x, g, w1, wb1, b1, w2, b2, wb)


def _k4_body(bus_ref, agg_ref, w1_ref, b1_ref, w2_ref, b2_ref, ws_ref, bs_ref,
             o_ref):
    pre = (_dot(bus_ref[...], w1_ref[...]) + agg_ref[:, :64] + b1_ref[...])
    h = jax.nn.relu(pre)
    bus_h = jax.nn.relu(_dot(h, w2_ref[...]) + b2_ref[...])
    s = _dot(bus_h, ws_ref[...]) + bs_ref[...]
    o_ref[...] = jnp.concatenate([s, bus_h], axis=1)


def _bus_mlp(bus, agg, w1, b1, w2, b2, ws, bs, bm=2000):
    n = bus.shape[0]
    return pl.pallas_call(
        _k4_body,
        grid=(n // bm,),
        in_specs=[_row(bm, 128), _row(bm, 128), _rep((128, 64)), _rep((1, 64)),
                  _rep((64, 64)), _rep((1, 64)), _rep((64, 64)), _rep((1, 64))],
        out_specs=_row(bm, 128),
        out_shape=jax.ShapeDtypeStruct((n, 128), jnp.float32),
        compiler_params=pltpu.CompilerParams(dimension_semantics=("parallel",)),
    )(bus, agg, w1, b1, w2, b2, ws, bs)


def _k5_body(g_ref, attr_ref, wba_ref, wbd_ref, w2b_ref, b2b_ref,
             wna_ref, wnd_ref, b1n_ref, w2n_ref, b2n_ref, bo_ref, bne_ref):
    attr = attr_ref[...]
    gs = g_ref[:, :64]
    gd = g_ref[:, 64:]
    h_b = jax.nn.relu(gs + _bdot(attr, wba_ref[...]) + _bdot(gd, wbd_ref[...]))
    bo_ref[...] = jax.nn.relu(_bdot(h_b, w2b_ref[...]) + b2b_ref[...])
    h_n = jax.nn.relu(_bdot(gd, wnd_ref[...]) + _bdot(attr, wna_ref[...])
                      + b1n_ref[...])
    bne = jax.nn.relu(_bdot(h_n, w2n_ref[...]) + b2n_ref[...])
    half = bne.shape[0] // 2
    bne_ref[...] = jnp.concatenate([bne[:half], bne[half:]], axis=1)


def _edge_mlp(g, attr, wba, wbd, w2b, b2b, wna, wnd, b1n, w2n, b2n, bm=2000):
    n = attr.shape[0]
    return pl.pallas_call(
        _k5_body,
        grid=(n // bm,),
        in_specs=[_row(bm, 128), _row(bm, 16), _rep((16, 64)),
                  _rep((64, 64)), _rep((64, 16)), _rep((1, 16)), _rep((16, 64)),
                  _rep((64, 64)), _rep((1, 64)), _rep((64, 64)), _rep((1, 64))],
        out_specs=[_row(bm, 16), _row(bm // 2, 128)],
        out_shape=[jax.ShapeDtypeStruct((n, 16), jnp.float32),
                   jax.ShapeDtypeStruct((n // 2, 128), jnp.float32)],
        compiler_params=pltpu.CompilerParams(dimension_semantics=("parallel",)),
    )(g, attr, wba, wbd, w2b, b2b, wna, wnd, b1n, w2n, b2n)


def _k6_body(bn_ref, o4_ref, wa_ref, wb_ref, b1_ref, w2_ref, b2_ref, o_ref):
    pre = (_dot(bn_ref[:, :64], wa_ref[...]) + _dot(o4_ref[:, 64:], wb_ref[...])
           + b1_ref[...])
    h = jax.nn.relu(pre)
    o_ref[...] = jax.nn.relu(_dot(h, w2_ref[...]) + b2_ref[...])


def _final_mlp(bn, o4, wa, wb, b1, w2, b2, bm=2000):
    n = bn.shape[0]
    return pl.pallas_call(
        _k6_body,
        grid=(n // bm,),
        in_specs=[_row(bm, 128), _row(bm, 128), _rep((64, 64)), _rep((64, 64)),
                  _rep((1, 64)), _rep((64, 128)), _rep((1, 128))],
        out_specs=_row(bm, 128),
        out_shape=jax.ShapeDtypeStruct((n, 128), jnp.float32),
        compiler_params=pltpu.CompilerParams(dimension_semantics=("parallel",)),
    )(bn, o4, wa, wb, b1, w2, b2)


_NC = 2
_NS = 16
_NW = _NC * _NS
_CH = 120
_KCH = 4
_SCH = _CH * _KCH


def _sc_mesh():
    return plsc.VectorSubcoreMesh(core_axis_name="c", subcore_axis_name="s",
                                  num_cores=_NC, num_subcores=_NS)


def _emit_gather_phase(table_hbm, idx_hbm, out_hbm, idx_v, rows_v, gsem, ssem,
                       base, s_count, tail, src_col, dst_col, width):
    def load_idx(slot, s):
        for j in range(_KCH):
            pltpu.sync_copy(
                idx_hbm.at[pl.ds(base + s * _SCH + j * _CH, _CH)],
                idx_v.at[slot, j, pl.ds(0, _CH)])

    def gathers(slot):
        return [pltpu.make_async_copy(
            table_hbm.at[idx_v.at[slot, j, pl.ds(0, _CH)]],
            rows_v.at[slot, pl.ds(j * _CH, _CH)],
            gsem.at[slot]) for j in range(_KCH)]

    def fire(slot, s):
        load_idx(slot, s)
        for c in gathers(slot):
            c.start()

    def wait_g(slot):
        for c in gathers(slot):
            c.wait()

    def store(slot, s, nrows=_SCH):
        return pltpu.make_async_copy(
            rows_v.at[slot, pl.ds(0, nrows), pl.ds(src_col, width)],
            out_hbm.at[pl.ds(base + s * _SCH, nrows), pl.ds(dst_col, width)],
            ssem.at[slot])

    fire(0, 0)
    fire(1, 1)
    wait_g(0)
    store(0, 0).start()

    s_even = s_count - (s_count % 2)

    @pl.loop(2, s_even, step=2)
    def _(cc):
        for slot in (0, 1):
            s = cc + slot
            store(slot, s).wait()
            fire(slot, s)
            wait_g(1 - slot)
            store(1 - slot, s - 1).start()

    if s_count % 2 == 1:
        s = s_count - 1
        store(0, s).wait()
        fire(0, s)
        wait_g(1)
        store(1, s - 1).start()
        wait_g(0)
        store(0, s).start()
        store(1, s - 1).wait()
        store(0, s).wait()
    elif tail:
        t0 = base + s_count * _SCH
        store(0, 0).wait()
        pltpu.sync_copy(idx_hbm.at[pl.ds(t0, tail)],
                        idx_v.at[0, 0, pl.ds(0, tail)])
        tg = pltpu.make_async_copy(
            table_hbm.at[idx_v.at[0, 0, pl.ds(0, tail)]],
            rows_v.at[0, pl.ds(0, tail)], gsem.at[0])
        tg.start()
        wait_g(1)
        store(1, s_count - 1).start()
        tg.wait()
        ts = pltpu.make_async_copy(
            rows_v.at[0, pl.ds(0, tail), pl.ds(src_col, width)],
            out_hbm.at[pl.ds(t0, tail), pl.ds(dst_col, width)],
            ssem.at[0])
        ts.start()
        store(1, s_count - 1).wait()
        ts.wait()
    else:
        wait_g(1)
        store(1, s_count - 1).start()
        store(0, 0).wait()
        store(1, s_count - 1).wait()


def _sc_gather_full(table, idx):
    b = idx.shape[0]
    n_per = b // _NW
    s_count = n_per // _SCH
    assert n_per % _SCH == 0 and b % _NW == 0

    @functools.partial(
        pl.kernel,
        out_type=jax.ShapeDtypeStruct((b, 128), jnp.float32),
        mesh=_sc_mesh(),
        scratch_types=[
            pltpu.VMEM((2, _KCH, 128), jnp.int32),
            pltpu.VMEM((2, _SCH, 128), jnp.float32),
            pltpu.SemaphoreType.DMA((2,)),
            pltpu.SemaphoreType.DMA((2,)),
        ],
        compiler_params=pltpu.CompilerParams(use_tc_tiling_on_sc=False))
    def k(table_hbm, idx_hbm, out_hbm, idx_v, rows_v, gsem, ssem):
        wid = lax.axis_index("s") * _NC + lax.axis_index("c")
        base = wid * n_per
        _emit_gather_phase(table_hbm, idx_hbm, out_hbm, idx_v, rows_v,
                           gsem, ssem, base, s_count, 0, 0, 0, 128)

    return k(table, idx)


def _sc_gather_edges(o4, src, dest):
    b = src.shape[0]
    n_per = b // _NW
    s_count = (n_per - 40) // _SCH
    assert n_per == s_count * _SCH + 40 and s_count % 2 == 0

    @functools.partial(
        pl.kernel,
        out_type=jax.ShapeDtypeStruct((b, 128), jnp.float32),
        mesh=_sc_mesh(),
        scratch_types=[
            pltpu.VMEM((2, _KCH, 128), jnp.int32),
            pltpu.VMEM((2, _SCH, 128), jnp.float32),
            pltpu.SemaphoreType.DMA((2,)),
            pltpu.SemaphoreType.DMA((2,)),
        ],
        compiler_params=pltpu.CompilerParams(use_tc_tiling_on_sc=False))
    def k(o4_hbm, src_hbm, dest_hbm, out_hbm, idx_v, rows_v, gsem, ssem):
        wid = lax.axis_index("s") * _NC + lax.axis_index("c")
        base = wid * n_per
        _emit_gather_phase(o4_hbm, src_hbm, out_hbm, idx_v, rows_v,
                           gsem, ssem, base, s_count, 40, 0, 0, 64)
        _emit_gather_phase(o4_hbm, dest_hbm, out_hbm, idx_v, rows_v,
                           gsem, ssem, base, s_count, 40, 64, 64, 64)

    return k(o4, src, dest)


def _sc_segsum(upd2, idx_ev, idx_od, n_out, tail=0):
    m = upd2.shape[0]
    rows_per = m // _NS
    nch = (rows_per - tail) // 64
    assert rows_per == nch * 64 + tail and nch % 2 == 0 and m % _NS == 0
    acc_rows = n_out + 48
    junk = n_out
    zrows = n_out // _NS

    @functools.partial(
        pl.kernel,
        out_type=jax.ShapeDtypeStruct((n_out, 128), jnp.float32),
        mesh=_sc_mesh(),
        scratch_types=[
            pltpu.VMEM((2, 2, 64), jnp.int32),
            pltpu.VMEM((2, 2, 64, 32), jnp.float32),
            pltpu.VMEM((128, 32), jnp.float32),
            pltpu.VMEM_SHARED((acc_rows, 32), jnp.float32),
            pltpu.SemaphoreType.DMA((2,)),
        ],
        compiler_params=pltpu.CompilerParams(use_tc_tiling_on_sc=False))
    def k(upd_hbm, ev_hbm, od_hbm, out_hbm, idx_v, upd_v, zbuf, acc, lsem):
        cid = lax.axis_index("c")
        sid = lax.axis_index("s")
        col0 = cid * 32
        base = sid * rows_per

        def dmas(slot, r0):
            return [
                pltpu.make_async_copy(ev_hbm.at[pl.ds(r0, 64)],
                                      idx_v.at[0, slot], lsem.at[slot]),
                pltpu.make_async_copy(od_hbm.at[pl.ds(r0, 64)],
                                      idx_v.at[1, slot], lsem.at[slot]),
                pltpu.make_async_copy(
                    upd_hbm.at[pl.ds(r0, 64), pl.ds(col0, 32)],
                    upd_v.at[0, slot], lsem.at[slot]),
                pltpu.make_async_copy(
                    upd_hbm.at[pl.ds(r0, 64), pl.ds(col0 + 64, 32)],
                    upd_v.at[1, slot], lsem.at[slot]),
            ]

        def fire(slot, r0):
            for c in dmas(slot, r0):
                c.start()

        def wait(slot, r0):
            for c in dmas(slot, r0):
                c.wait()

        def scatter(slot):
            pltpu.sync_copy(upd_v.at[0, slot], acc.at[idx_v.at[0, slot]],
                            add=True)
            pltpu.sync_copy(upd_v.at[1, slot], acc.at[idx_v.at[1, slot]],
                            add=True)

        @pl.loop(0, 128)
        def _(i):
            zbuf[i, pl.ds(0, 16)] = jnp.zeros((16,), jnp.float32)
            zbuf[i, pl.ds(16, 16)] = jnp.zeros((16,), jnp.float32)
        nz_full, z_rem = divmod(zrows, 128)
        for j in range(nz_full):
            pltpu.sync_copy(zbuf, acc.at[pl.ds(sid * zrows + j * 128, 128)])
        if z_rem:
            pltpu.sync_copy(zbuf.at[pl.ds(0, z_rem)],
                            acc.at[pl.ds(sid * zrows + nz_full * 128, z_rem)])
        plsc.subcore_barrier()

        fire(0, base)
        fire(1, base + 64)

        @pl.loop(0, nch - 2, step=2)
        def _(cc):
            for slot in (0, 1):
                r0 = base + (cc + slot) * 64
                wait(slot, r0)
                scatter(slot)
                fire(slot, r0 + 128)

        for slot in (0, 1):
            r0 = base + (nch - 2 + slot) * 64
            wait(slot, r0)
            scatter(slot)

        if tail:
            r0 = base + nch * 64
            for kk in range(0, 64, 16):
                idx_v[0, 0, pl.ds(kk, 16)] = jnp.full((16,), junk, jnp.int32)
                idx_v[1, 0, pl.ds(kk, 16)] = jnp.full((16,), junk, jnp.int32)
            pltpu.sync_copy(ev_hbm.at[pl.ds(r0, tail)],
                            idx_v.at[0, 0, pl.ds(0, tail)])
            pltpu.sync_copy(od_hbm.at[pl.ds(r0, tail)],
                            idx_v.at[1, 0, pl.ds(0, tail)])
            pltpu.sync_copy(upd_hbm.at[pl.ds(r0, tail), pl.ds(col0, 32)],
                            upd_v.at[0, 0, pl.ds(0, tail)])
            pltpu.sync_copy(upd_hbm.at[pl.ds(r0, tail), pl.ds(col0 + 64, 32)],
                            upd_v.at[1, 0, pl.ds(0, tail)])
            scatter(0)

        plsc.subcore_barrier()

        pltpu.sync_copy(acc.at[pl.ds(sid * zrows, zrows)],
                        out_hbm.at[pl.ds(sid * zrows, zrows), pl.ds(col0, 32)])

    return k(upd2, idx_ev, idx_od)


def kernel(bus, shunt, gen, load, branch_attr, shunt_to_bus, gen_to_bus,
           load_to_bus, branch_index, p_bus, p_shunt, p_gen, p_load, p_branch,
           p_bb, p_bn):
    n_bus = bus.shape[0]
    n_shunt = shunt.shape[0]
    n_gen = gen.shape[0]

    n_items = n_shunt + n_gen + load.shape[0]
    b_items = _NW * _SCH * 5
    idx_items = jnp.concatenate([
        shunt_to_bus, gen_to_bus, load_to_bus,
        jnp.arange(b_items - n_items, dtype=jnp.int32) % n_bus])
    g_bus = _sc_gather_full(bus, idx_items)

    w1b = p_bus['W1']
    shunt_o, u_s = _item_mlp(shunt, g_bus, 0, p_shunt['W1'][:16],
                             p_shunt['W1'][16:], p_shunt['b1'][None],
                             p_shunt['W2'], p_shunt['b2'][None], w1b[128:192])
    gen_o, u_g = _item_mlp(gen, g_bus, n_shunt, p_gen['W1'][:16],
                           p_gen['W1'][16:], p_gen['b1'][None],
                           p_gen['W2'], p_gen['b2'][None], w1b[192:256])
    load_o, u_l = _item_mlp(load, g_bus, n_shunt + n_gen, p_load['W1'][:16],
                            p_load['W1'][16:], p_load['b1'][None],
                            p_load['W2'], p_load['b2'][None], w1b[256:320])

    m_items = n_items // 2
    m_pad = 36864
    upd2 = jnp.concatenate(
        [u_s, u_g, u_l,
         jnp.zeros((m_pad - m_items, 128), jnp.float32)], axis=0)
    idx = jnp.concatenate([shunt_to_bus, gen_to_bus, load_to_bus])
    idx2 = idx.reshape(-1, 2, 1000)
    pad_i = jnp.full((m_pad - m_items,), n_bus, jnp.int32)
    idx_ev = jnp.concatenate([idx2[:, 0].ravel(), pad_i])
    idx_od = jnp.concatenate([idx2[:, 1].ravel(), pad_i])
    agg = _sc_segsum(upd2, idx_ev, idx_od, n_bus)

    o4 = _bus_mlp(bus, agg, w1b[:128], p_bus['b1'][None], p_bus['W2'],
                  p_bus['b2'][None], p_branch['W1'][:64], p_branch['b1'][None])

    src = branch_index[0]
    dest = branch_index[1]
    g_edge = _sc_gather_edges(o4, src, dest)

    w1br = p_branch['W1']
    w1n = p_bb['W1']
    branch_o, bn_e2 = _edge_mlp(
        g_edge, branch_attr, w1br[64:80], w1br[80:144], p_branch['W2'],
        p_branch['b2'][None], w1n[64:80], w1n[:64], p_bb['b1'][None],
        p_bb['W2'], p_bb['b2'][None])

    src2 = src.reshape(-1, 2, 1000)
    bn = _sc_segsum(bn_e2, src2[:, 0].ravel(), src2[:, 1].ravel(), n_bus,
                    tail=40)

    w1f = p_bn['W1']
    bus_out = _final_mlp(bn, o4, w1f[:64], w1f[64:128], p_bn['b1'][None],
                         p_bn['W2'], p_bn['b2'][None])

    return (bus_out, shunt_o, gen_o, load_o, branch_o)


# speedup vs baseline: 1.3300x; 1.3300x over previous
import functools

import jax
import jax.numpy as jnp
from jax import lax
from jax.experimental import pallas as pl
from jax.experimental.pallas import tpu as pltpu
from jax.experimental.pallas import tpu_sc as plsc


def _rep(shape):
    return pl.BlockSpec(shape, lambda *_: (0,) * len(shape))


def _row(bm, cols):
    return pl.BlockSpec((bm, cols), lambda i: (i, 0))


def _dot(a, b):
    return jnp.dot(a, b, preferred_element_type=jnp.float32)


def _bdot(a, b):
    return jnp.dot(a.astype(jnp.bfloat16), b.astype(jnp.bfloat16),
                   preferred_element_type=jnp.float32)


def _dot_t(a_t, b):
    return lax.dot_general(a_t.astype(jnp.bfloat16), b.astype(jnp.bfloat16),
                           (((0,), (0,)), ((), ())),
                           preferred_element_type=jnp.float32)


def _dot_ot(w, h):
    return lax.dot_general(w.astype(jnp.bfloat16), h.astype(jnp.bfloat16),
                           (((0,), (1,)), ((), ())),
                           preferred_element_type=jnp.float32)


def _k2_body(x_ref, g_ref, w1_ref, wb1_ref, b1_ref, w2_ref, b2_ref, wb_ref,
             o_ref, u_ref):
    h = jax.nn.relu(_bdot(x_ref[...], w1_ref[...])
                    + _bdot(g_ref[...], wb1_ref[...]) + b1_ref[...])
    o = jax.nn.relu(_bdot(h, w2_ref[...]) + b2_ref[...])
    o_ref[...] = o
    u = _bdot(o, wb_ref[...])
    half = u.shape[0] // 2
    u_ref[...] = jnp.concatenate([u[:half], u[half:]], axis=1)


def _item_mlp(x, g, g_off, w1, wb1, b1, w2, b2, wb, bm=2000):
    n = x.shape[0]
    ob = g_off // bm
    return pl.pallas_call(
        _k2_body,
        grid=(n // bm,),
        in_specs=[_row(bm, 16),
                  pl.BlockSpec((bm, 128), lambda i: (i + ob, 0)),
                  _rep((16, 64)), _rep((128, 64)), _rep((1, 64)),
                  _rep((64, 64)), _rep((1, 64)), _rep((64, 64))],
        out_specs=[_row(bm, 64), _row(bm // 2, 128)],
        out_shape=[jax.ShapeDtypeStruct((n, 64), jnp.float32),
                   jax.ShapeDtypeStruct((n // 2, 128), jnp.float32)],
        compiler_params=pltpu.CompilerParams(dimension_semantics=("parallel",)),
    )(x, g, w1, wb1, b1, w2, b2, wb)


def _k4_body(bus_ref, agg_ref, w1_ref, b1_ref, w2_ref, b2_ref, ws_ref, bs_ref,
             o_ref):
    pre = (_dot(bus_ref[...], w1_ref[...]) + agg_ref[:, :64] + b1_ref[...])
    h = jax.nn.relu(pre)
    bus_h = jax.nn.relu(_dot(h, w2_ref[...]) + b2_ref[...])
    s = _dot(bus_h, ws_ref[...]) + bs_ref[...]
    o_ref[...] = jnp.concatenate([s, bus_h], axis=1)


def _bus_mlp(bus, agg, w1, b1, w2, b2, ws, bs, bm=2000):
    n = bus.shape[0]
    return pl.pallas_call(
        _k4_body,
        grid=(n // bm,),
        in_specs=[_row(bm, 128), _row(bm, 128), _rep((128, 64)), _rep((1, 64)),
                  _rep((64, 64)), _rep((1, 64)), _rep((64, 64)), _rep((1, 64))],
        out_specs=_row(bm, 128),
        out_shape=jax.ShapeDtypeStruct((n, 128), jnp.float32),
        compiler_params=pltpu.CompilerParams(dimension_semantics=("parallel",)),
    )(bus, agg, w1, b1, w2, b2, ws, bs)


def _k5_body(g_ref, attrt_ref, wba_ref, wbd_ref, w2b_ref, b2bt_ref,
             wna_ref, wnd_ref, b1n_ref, w2n_ref, b2n_ref, bot_ref, bne_ref):
    attrt = attrt_ref[...]
    gs = g_ref[:, :64]
    gd = g_ref[:, 64:]
    h_b = jax.nn.relu(gs + _dot_t(attrt, wba_ref[...])
                      + _bdot(gd, wbd_ref[...]))
    bot_ref[...] = jax.nn.relu(_dot_ot(w2b_ref[...], h_b) + b2bt_ref[...])
    h_n = jax.nn.relu(_bdot(gd, wnd_ref[...]) + _dot_t(attrt, wna_ref[...])
                      + b1n_ref[...])
    bne = jax.nn.relu(_bdot(h_n, w2n_ref[...]) + b2n_ref[...])
    half = bne.shape[0] // 2
    bne_ref[...] = jnp.concatenate([bne[:half], bne[half:]], axis=1)


def _edge_mlp(g, attr_t, wba, wbd, w2b, b2bt, wna, wnd, b1n, w2n, b2n, bm=3200):
    n = attr_t.shape[1]
    bo_t, bne = pl.pallas_call(
        _k5_body,
        grid=(n // bm,),
        in_specs=[_row(bm, 128),
                  pl.BlockSpec((16, bm), lambda i: (0, i)),
                  _rep((16, 64)),
                  _rep((64, 64)), _rep((64, 16)), _rep((16, 1)), _rep((16, 64)),
                  _rep((64, 64)), _rep((1, 64)), _rep((64, 64)), _rep((1, 64))],
        out_specs=[pl.BlockSpec((16, bm), lambda i: (0, i)),
                   _row(bm // 2, 128)],
        out_shape=[jax.ShapeDtypeStruct((16, n), jnp.float32),
                   jax.ShapeDtypeStruct((n // 2, 128), jnp.float32)],
        compiler_params=pltpu.CompilerParams(dimension_semantics=("parallel",)),
    )(g, attr_t, wba, wbd, w2b, b2bt, wna, wnd, b1n, w2n, b2n)
    return bo_t.T, bne


def _k6_body(bn_ref, o4_ref, wa_ref, wb_ref, b1_ref, w2_ref, b2_ref, o_ref):
    pre = (_dot(bn_ref[:, :64], wa_ref[...]) + _dot(o4_ref[:, 64:], wb_ref[...])
           + b1_ref[...])
    h = jax.nn.relu(pre)
    o_ref[...] = jax.nn.relu(_dot(h, w2_ref[...]) + b2_ref[...])


def _final_mlp(bn, o4, wa, wb, b1, w2, b2, bm=2000):
    n = bn.shape[0]
    return pl.pallas_call(
        _k6_body,
        grid=(n // bm,),
        in_specs=[_row(bm, 128), _row(bm, 128), _rep((64, 64)), _rep((64, 64)),
                  _rep((1, 64)), _rep((64, 128)), _rep((1, 128))],
        out_specs=_row(bm, 128),
        out_shape=jax.ShapeDtypeStruct((n, 128), jnp.float32),
        compiler_params=pltpu.CompilerParams(dimension_semantics=("parallel",)),
    )(bn, o4, wa, wb, b1, w2, b2)


_NC = 2
_NS = 16
_NW = _NC * _NS
_CH = 120
_KCH = 4
_SCH = _CH * _KCH


def _sc_mesh():
    return plsc.VectorSubcoreMesh(core_axis_name="c", subcore_axis_name="s",
                                  num_cores=_NC, num_subcores=_NS)


def _emit_gather_phase(table_hbm, idx_hbm, out_hbm, idx_v, rows_v, gsem, ssem,
                       base, s_count, tail, src_col, dst_col, width):
    def load_idx(slot, s):
        for j in range(_KCH):
            pltpu.sync_copy(
                idx_hbm.at[pl.ds(base + s * _SCH + j * _CH, _CH)],
                idx_v.at[slot, j, pl.ds(0, _CH)])

    def gathers(slot):
        return [pltpu.make_async_copy(
            table_hbm.at[idx_v.at[slot, j, pl.ds(0, _CH)]],
            rows_v.at[slot, pl.ds(j * _CH, _CH)],
            gsem.at[slot]) for j in range(_KCH)]

    def fire(slot, s):
        load_idx(slot, s)
        for c in gathers(slot):
            c.start()

    def wait_g(slot):
        for c in gathers(slot):
            c.wait()

    def store(slot, s, nrows=_SCH):
        return pltpu.make_async_copy(
            rows_v.at[slot, pl.ds(0, nrows), pl.ds(src_col, width)],
            out_hbm.at[pl.ds(base + s * _SCH, nrows), pl.ds(dst_col, width)],
            ssem.at[slot])

    fire(0, 0)
    fire(1, 1)
    wait_g(0)
    store(0, 0).start()

    s_even = s_count - (s_count % 2)

    @pl.loop(2, s_even, step=2)
    def _(cc):
        for slot in (0, 1):
            s = cc + slot
            store(slot, s).wait()
            fire(slot, s)
            wait_g(1 - slot)
            store(1 - slot, s - 1).start()

    if s_count % 2 == 1:
        s = s_count - 1
        store(0, s).wait()
        fire(0, s)
        wait_g(1)
        store(1, s - 1).start()
        wait_g(0)
        store(0, s).start()
        store(1, s - 1).wait()
        store(0, s).wait()
    elif tail:
        t0 = base + s_count * _SCH
        store(0, 0).wait()
        pltpu.sync_copy(idx_hbm.at[pl.ds(t0, tail)],
                        idx_v.at[0, 0, pl.ds(0, tail)])
        tg = pltpu.make_async_copy(
            table_hbm.at[idx_v.at[0, 0, pl.ds(0, tail)]],
            rows_v.at[0, pl.ds(0, tail)], gsem.at[0])
        tg.start()
        wait_g(1)
        store(1, s_count - 1).start()
        tg.wait()
        ts = pltpu.make_async_copy(
            rows_v.at[0, pl.ds(0, tail), pl.ds(src_col, width)],
            out_hbm.at[pl.ds(t0, tail), pl.ds(dst_col, width)],
            ssem.at[0])
        ts.start()
        store(1, s_count - 1).wait()
        ts.wait()
    else:
        wait_g(1)
        store(1, s_count - 1).start()
        store(0, 0).wait()
        store(1, s_count - 1).wait()


def _sc_gather_full(table, idx):
    b = idx.shape[0]
    n_per = b // _NW
    s_count = n_per // _SCH
    assert n_per % _SCH == 0 and b % _NW == 0

    @functools.partial(
        pl.kernel,
        out_type=jax.ShapeDtypeStruct((b, 128), jnp.float32),
        mesh=_sc_mesh(),
        scratch_types=[
            pltpu.VMEM((2, _KCH, 128), jnp.int32),
            pltpu.VMEM((2, _SCH, 128), jnp.float32),
            pltpu.SemaphoreType.DMA((2,)),
            pltpu.SemaphoreType.DMA((2,)),
        ],
        compiler_params=pltpu.CompilerParams(use_tc_tiling_on_sc=False))
    def k(table_hbm, idx_hbm, out_hbm, idx_v, rows_v, gsem, ssem):
        wid = lax.axis_index("s") * _NC + lax.axis_index("c")
        base = wid * n_per
        _emit_gather_phase(table_hbm, idx_hbm, out_hbm, idx_v, rows_v,
                           gsem, ssem, base, s_count, 0, 0, 0, 128)

    return k(table, idx)


def _sc_gather_edges(o4, src, dest):
    b = src.shape[0]
    n_per = b // _NW
    s_count = (n_per - 40) // _SCH
    assert n_per == s_count * _SCH + 40 and s_count % 2 == 0

    @functools.partial(
        pl.kernel,
        out_type=jax.ShapeDtypeStruct((b, 128), jnp.float32),
        mesh=_sc_mesh(),
        scratch_types=[
            pltpu.VMEM((2, _KCH, 128), jnp.int32),
            pltpu.VMEM((2, _SCH, 128), jnp.float32),
            pltpu.SemaphoreType.DMA((2,)),
            pltpu.SemaphoreType.DMA((2,)),
        ],
        compiler_params=pltpu.CompilerParams(use_tc_tiling_on_sc=False))
    def k(o4_hbm, src_hbm, dest_hbm, out_hbm, idx_v, rows_v, gsem, ssem):
        wid = lax.axis_index("s") * _NC + lax.axis_index("c")
        base = wid * n_per
        _emit_gather_phase(o4_hbm, src_hbm, out_hbm, idx_v, rows_v,
                           gsem, ssem, base, s_count, 40, 0, 0, 64)
        _emit_gather_phase(o4_hbm, dest_hbm, out_hbm, idx_v, rows_v,
                           gsem, ssem, base, s_count, 40, 64, 64, 64)

    return k(o4, src, dest)


def _sc_segsum(upd2, idx_ev, idx_od, n_out, tail=0):
    m = upd2.shape[0]
    rows_per = m // _NS
    nch = (rows_per - tail) // 64
    assert rows_per == nch * 64 + tail and nch % 2 == 0 and m % _NS == 0
    acc_rows = n_out + 48
    junk = n_out
    zrows = n_out // _NS

    @functools.partial(
        pl.kernel,
        out_type=jax.ShapeDtypeStruct((n_out, 128), jnp.float32),
        mesh=_sc_mesh(),
        scratch_types=[
            pltpu.VMEM((2, 2, 64), jnp.int32),
            pltpu.VMEM((2, 2, 64, 32), jnp.float32),
            pltpu.VMEM((128, 32), jnp.float32),
            pltpu.VMEM_SHARED((acc_rows, 32), jnp.float32),
            pltpu.SemaphoreType.DMA((2,)),
        ],
        compiler_params=pltpu.CompilerParams(use_tc_tiling_on_sc=False))
    def k(upd_hbm, ev_hbm, od_hbm, out_hbm, idx_v, upd_v, zbuf, acc, lsem):
        cid = lax.axis_index("c")
        sid = lax.axis_index("s")
        col0 = cid * 32
        base = sid * rows_per

        def dmas(slot, r0):
            return [
                pltpu.make_async_copy(ev_hbm.at[pl.ds(r0, 64)],
                                      idx_v.at[0, slot], lsem.at[slot]),
                pltpu.make_async_copy(od_hbm.at[pl.ds(r0, 64)],
                                      idx_v.at[1, slot], lsem.at[slot]),
                pltpu.make_async_copy(
                    upd_hbm.at[pl.ds(r0, 64), pl.ds(col0, 32)],
                    upd_v.at[0, slot], lsem.at[slot]),
                pltpu.make_async_copy(
                    upd_hbm.at[pl.ds(r0, 64), pl.ds(col0 + 64, 32)],
                    upd_v.at[1, slot], lsem.at[slot]),
            ]

        def fire(slot, r0):
            for c in dmas(slot, r0):
                c.start()

        def wait(slot, r0):
            for c in dmas(slot, r0):
                c.wait()

        def scatter(slot):
            pltpu.sync_copy(upd_v.at[0, slot], acc.at[idx_v.at[0, slot]],
                            add=True)
            pltpu.sync_copy(upd_v.at[1, slot], acc.at[idx_v.at[1, slot]],
                            add=True)

        @pl.loop(0, 128)
        def _(i):
            zbuf[i, pl.ds(0, 16)] = jnp.zeros((16,), jnp.float32)
            zbuf[i, pl.ds(16, 16)] = jnp.zeros((16,), jnp.float32)
        nz_full, z_rem = divmod(zrows, 128)
        for j in range(nz_full):
            pltpu.sync_copy(zbuf, acc.at[pl.ds(sid * zrows + j * 128, 128)])
        if z_rem:
            pltpu.sync_copy(zbuf.at[pl.ds(0, z_rem)],
                            acc.at[pl.ds(sid * zrows + nz_full * 128, z_rem)])
        plsc.subcore_barrier()

        fire(0, base)
        fire(1, base + 64)

        @pl.loop(0, nch - 2, step=2)
        def _(cc):
            for slot in (0, 1):
                r0 = base + (cc + slot) * 64
                wait(slot, r0)
                scatter(slot)
                fire(slot, r0 + 128)

        for slot in (0, 1):
            r0 = base + (nch - 2 + slot) * 64
            wait(slot, r0)
            scatter(slot)

        if tail:
            r0 = base + nch * 64
            for kk in range(0, 64, 16):
                idx_v[0, 0, pl.ds(kk, 16)] = jnp.full((16,), junk, jnp.int32)
                idx_v[1, 0, pl.ds(kk, 16)] = jnp.full((16,), junk, jnp.int32)
            pltpu.sync_copy(ev_hbm.at[pl.ds(r0, tail)],
                            idx_v.at[0, 0, pl.ds(0, tail)])
            pltpu.sync_copy(od_hbm.at[pl.ds(r0, tail)],
                            idx_v.at[1, 0, pl.ds(0, tail)])
            pltpu.sync_copy(upd_hbm.at[pl.ds(r0, tail), pl.ds(col0, 32)],
                            upd_v.at[0, 0, pl.ds(0, tail)])
            pltpu.sync_copy(upd_hbm.at[pl.ds(r0, tail), pl.ds(col0 + 64, 32)],
                            upd_v.at[1, 0, pl.ds(0, tail)])
            scatter(0)

        plsc.subcore_barrier()

        pltpu.sync_copy(acc.at[pl.ds(sid * zrows, zrows)],
                        out_hbm.at[pl.ds(sid * zrows, zrows), pl.ds(col0, 32)])

    return k(upd2, idx_ev, idx_od)


def kernel(bus, shunt, gen, load, branch_attr, shunt_to_bus, gen_to_bus,
           load_to_bus, branch_index, p_bus, p_shunt, p_gen, p_load, p_branch,
           p_bb, p_bn):
    n_bus = bus.shape[0]
    n_shunt = shunt.shape[0]
    n_gen = gen.shape[0]

    n_items = n_shunt + n_gen + load.shape[0]
    b_items = _NW * _SCH * 5
    idx_items = jnp.concatenate([
        shunt_to_bus, gen_to_bus, load_to_bus,
        jnp.arange(b_items - n_items, dtype=jnp.int32) % n_bus])
    g_bus = _sc_gather_full(bus, idx_items)

    w1b = p_bus['W1']
    shunt_o, u_s = _item_mlp(shunt, g_bus, 0, p_shunt['W1'][:16],
                             p_shunt['W1'][16:], p_shunt['b1'][None],
                             p_shunt['W2'], p_shunt['b2'][None], w1b[128:192])
    gen_o, u_g = _item_mlp(gen, g_bus, n_shunt, p_gen['W1'][:16],
                           p_gen['W1'][16:], p_gen['b1'][None],
                           p_gen['W2'], p_gen['b2'][None], w1b[192:256])
    load_o, u_l = _item_mlp(load, g_bus, n_shunt + n_gen, p_load['W1'][:16],
                            p_load['W1'][16:], p_load['b1'][None],
                            p_load['W2'], p_load['b2'][None], w1b[256:320])

    m_items = n_items // 2
    m_pad = 36864
    upd2 = jnp.concatenate(
        [u_s, u_g, u_l,
         jnp.zeros((m_pad - m_items, 128), jnp.float32)], axis=0)
    idx = jnp.concatenate([shunt_to_bus, gen_to_bus, load_to_bus])
    idx2 = idx.reshape(-1, 2, 1000)
    pad_i = jnp.full((m_pad - m_items,), n_bus, jnp.int32)
    idx_ev = jnp.concatenate([idx2[:, 0].ravel(), pad_i])
    idx_od = jnp.concatenate([idx2[:, 1].ravel(), pad_i])
    agg = _sc_segsum(upd2, idx_ev, idx_od, n_bus)

    o4 = _bus_mlp(bus, agg, w1b[:128], p_bus['b1'][None], p_bus['W2'],
                  p_bus['b2'][None], p_branch['W1'][:64], p_branch['b1'][None])

    src = branch_index[0]
    dest = branch_index[1]
    g_edge = _sc_gather_edges(o4, src, dest)

    w1br = p_branch['W1']
    w1n = p_bb['W1']
    branch_o, bn_e2 = _edge_mlp(
        g_edge, branch_attr.T, w1br[64:80], w1br[80:144], p_branch['W2'],
        p_branch['b2'][:, None], w1n[64:80], w1n[:64], p_bb['b1'][None],
        p_bb['W2'], p_bb['b2'][None])

    src2 = src.reshape(-1, 2, 1600)
    bn = _sc_segsum(bn_e2, src2[:, 0].ravel(), src2[:, 1].ravel(), n_bus,
                    tail=40)

    w1f = p_bn['W1']
    bus_out = _final_mlp(bn, o4, w1f[:64], w1f[64:128], p_bn['b1'][None],
                         p_bn['W2'], p_bn['b2'][None])

    return (bus_out, shunt_o, gen_o, load_o, branch_o)


# speedup vs baseline: 1.5691x; 1.1798x over previous
import functools

import jax
import jax.numpy as jnp
from jax import lax
from jax.experimental import pallas as pl
from jax.experimental.pallas import tpu as pltpu
from jax.experimental.pallas import tpu_sc as plsc


def _rep(shape):
    return pl.BlockSpec(shape, lambda *_: (0,) * len(shape))


def _row(bm, cols):
    return pl.BlockSpec((bm, cols), lambda i: (i, 0))


def _dot(a, b):
    return jnp.dot(a, b, preferred_element_type=jnp.float32)


def _bdot(a, b):
    return jnp.dot(a.astype(jnp.bfloat16), b.astype(jnp.bfloat16),
                   preferred_element_type=jnp.float32)


def _dot_t(a_t, b):
    return lax.dot_general(a_t.astype(jnp.bfloat16), b.astype(jnp.bfloat16),
                           (((0,), (0,)), ((), ())),
                           preferred_element_type=jnp.float32)


def _dot_ot(w, h):
    return lax.dot_general(w.astype(jnp.bfloat16), h.astype(jnp.bfloat16),
                           (((0,), (1,)), ((), ())),
                           preferred_element_type=jnp.float32)


def _k2_body(x_ref, g_ref, w1_ref, wb1_ref, b1_ref, w2_ref, b2_ref, wb_ref,
             o_ref, u_ref):
    h = jax.nn.relu(_bdot(x_ref[...], w1_ref[...])
                    + _bdot(g_ref[...], wb1_ref[...]) + b1_ref[...])
    o = jax.nn.relu(_bdot(h, w2_ref[...]) + b2_ref[...])
    o_ref[...] = o
    u = _bdot(o, wb_ref[...])
    half = u.shape[0] // 2
    u_ref[...] = jnp.concatenate([u[:half], u[half:]], axis=1)


def _item_mlp(x, g, g_off, w1, wb1, b1, w2, b2, wb, bm=2000):
    n = x.shape[0]
    ob = g_off // bm
    return pl.pallas_call(
        _k2_body,
        grid=(n // bm,),
        in_specs=[_row(bm, 16),
                  pl.BlockSpec((bm, 128), lambda i: (i + ob, 0)),
                  _rep((16, 64)), _rep((128, 64)), _rep((1, 64)),
                  _rep((64, 64)), _rep((1, 64)), _rep((64, 64))],
        out_specs=[_row(bm, 64), _row(bm // 2, 128)],
        out_shape=[jax.ShapeDtypeStruct((n, 64), jnp.float32),
                   jax.ShapeDtypeStruct((n // 2, 128), jnp.float32)],
        compiler_params=pltpu.CompilerParams(dimension_semantics=("parallel",)),
    )(x, g, w1, wb1, b1, w2, b2, wb)


def _k4_body(bus_ref, agg_ref, w1_ref, b1_ref, w2_ref, b2_ref, ws_ref, bs_ref,
             o_ref):
    pre = (_dot(bus_ref[...], w1_ref[...]) + agg_ref[:, :64] + b1_ref[...])
    h = jax.nn.relu(pre)
    bus_h = jax.nn.relu(_dot(h, w2_ref[...]) + b2_ref[...])
    s = _dot(bus_h, ws_ref[...]) + bs_ref[...]
    o_ref[...] = jnp.concatenate([s, bus_h], axis=1)


def _bus_mlp(bus, agg, w1, b1, w2, b2, ws, bs, bm=2000):
    n = bus.shape[0]
    return pl.pallas_call(
        _k4_body,
        grid=(n // bm,),
        in_specs=[_row(bm, 128), _row(bm, 128), _rep((128, 64)), _rep((1, 64)),
                  _rep((64, 64)), _rep((1, 64)), _rep((64, 64)), _rep((1, 64))],
        out_specs=_row(bm, 128),
        out_shape=jax.ShapeDtypeStruct((n, 128), jnp.float32),
        compiler_params=pltpu.CompilerParams(dimension_semantics=("parallel",)),
    )(bus, agg, w1, b1, w2, b2, ws, bs)


def _k5_body(g_ref, attrt_ref, wba_ref, wbd_ref, w2b_ref, b2bt_ref,
             wna_ref, wnd_ref, b1n_ref, w2n_ref, b2n_ref, bot_ref, bne_ref):
    attrt = attrt_ref[...]
    gs = g_ref[:, :64]
    gd = g_ref[:, 64:]
    h_b = jax.nn.relu(gs + _dot_t(attrt, wba_ref[...])
                      + _bdot(gd, wbd_ref[...]))
    bot_ref[...] = jax.nn.relu(_dot_ot(w2b_ref[...], h_b) + b2bt_ref[...])
    h_n = jax.nn.relu(_bdot(gd, wnd_ref[...]) + _dot_t(attrt, wna_ref[...])
                      + b1n_ref[...])
    bne = jax.nn.relu(_bdot(h_n, w2n_ref[...]) + b2n_ref[...])
    half = bne.shape[0] // 2
    bne_ref[...] = jnp.concatenate([bne[:half], bne[half:]], axis=1)


def _edge_mlp(g, attr_t, e_off, wba, wbd, w2b, b2bt, wna, wnd, b1n, w2n, b2n,
              bm=3200):
    n = g.shape[0]
    ob = e_off // bm
    bo_t, bne = pl.pallas_call(
        _k5_body,
        grid=(n // bm,),
        in_specs=[_row(bm, 128),
                  pl.BlockSpec((16, bm), lambda i: (0, i + ob)),
                  _rep((16, 64)),
                  _rep((64, 64)), _rep((64, 16)), _rep((16, 1)), _rep((16, 64)),
                  _rep((64, 64)), _rep((1, 64)), _rep((64, 64)), _rep((1, 64))],
        out_specs=[pl.BlockSpec((16, bm), lambda i: (0, i)),
                   _row(bm // 2, 128)],
        out_shape=[jax.ShapeDtypeStruct((16, n), jnp.float32),
                   jax.ShapeDtypeStruct((n // 2, 128), jnp.float32)],
        compiler_params=pltpu.CompilerParams(dimension_semantics=("parallel",)),
    )(g, attr_t, wba, wbd, w2b, b2bt, wna, wnd, b1n, w2n, b2n)
    return bo_t, bne


def _k6_body(bn1_ref, bn2_ref, o4_ref, wa_ref, wb_ref, b1_ref, w2_ref, b2_ref,
             o_ref):
    bn = bn1_ref[:, :64] + bn2_ref[:, :64]
    pre = (_bdot(bn, wa_ref[...]) + _bdot(o4_ref[:, 64:], wb_ref[...])
           + b1_ref[...])
    h = jax.nn.relu(pre)
    o_ref[...] = jax.nn.relu(_bdot(h, w2_ref[...]) + b2_ref[...])


def _final_mlp(bn1, bn2, o4, wa, wb, b1, w2, b2, bm=2000):
    n = bn1.shape[0]
    return pl.pallas_call(
        _k6_body,
        grid=(n // bm,),
        in_specs=[_row(bm, 128), _row(bm, 128), _row(bm, 128),
                  _rep((64, 64)), _rep((64, 64)),
                  _rep((1, 64)), _rep((64, 128)), _rep((1, 128))],
        out_specs=_row(bm, 128),
        out_shape=jax.ShapeDtypeStruct((n, 128), jnp.float32),
        compiler_params=pltpu.CompilerParams(dimension_semantics=("parallel",)),
    )(bn1, bn2, o4, wa, wb, b1, w2, b2)


_NC = 2
_NS = 16
_NW = _NC * _NS
_CH = 120
_KCH = 4
_SCH = _CH * _KCH


def _sc_mesh():
    return plsc.VectorSubcoreMesh(core_axis_name="c", subcore_axis_name="s",
                                  num_cores=_NC, num_subcores=_NS)


def _emit_gather_phase(table_hbm, idx_hbm, out_hbm, idx_v, rows_v, gsem, ssem,
                       base, s_count, tail, src_col, dst_col, width):
    def load_idx(slot, s):
        for j in range(_KCH):
            pltpu.sync_copy(
                idx_hbm.at[pl.ds(base + s * _SCH + j * _CH, _CH)],
                idx_v.at[slot, j, pl.ds(0, _CH)])

    def gathers(slot):
        return [pltpu.make_async_copy(
            table_hbm.at[idx_v.at[slot, j, pl.ds(0, _CH)]],
            rows_v.at[slot, pl.ds(j * _CH, _CH)],
            gsem.at[slot]) for j in range(_KCH)]

    def fire(slot, s):
        load_idx(slot, s)
        for c in gathers(slot):
            c.start()

    def wait_g(slot):
        for c in gathers(slot):
            c.wait()

    def store(slot, s, nrows=_SCH):
        return pltpu.make_async_copy(
            rows_v.at[slot, pl.ds(0, nrows), pl.ds(src_col, width)],
            out_hbm.at[pl.ds(base + s * _SCH, nrows), pl.ds(dst_col, width)],
            ssem.at[slot])

    def tail_gathers(slot):
        cs = []
        off = 0
        for j, tc in enumerate(tail):
            cs.append(pltpu.make_async_copy(
                table_hbm.at[idx_v.at[slot, j, pl.ds(0, tc)]],
                rows_v.at[slot, pl.ds(off, tc)], gsem.at[slot]))
            off += tc
        return cs

    def fire_tail(slot):
        t0 = base + s_count * _SCH
        off = 0
        for j, tc in enumerate(tail):
            pltpu.sync_copy(idx_hbm.at[pl.ds(t0 + off, tc)],
                            idx_v.at[slot, j, pl.ds(0, tc)])
            off += tc
        for c in tail_gathers(slot):
            c.start()

    def tail_store(slot):
        t0 = base + s_count * _SCH
        trows = sum(tail)
        return pltpu.make_async_copy(
            rows_v.at[slot, pl.ds(0, trows), pl.ds(src_col, width)],
            out_hbm.at[pl.ds(t0, trows), pl.ds(dst_col, width)],
            ssem.at[slot])

    fire(0, 0)
    fire(1, 1)
    wait_g(0)
    store(0, 0).start()

    s_even = s_count - (s_count % 2)

    @pl.loop(2, s_even, step=2)
    def _(cc):
        for slot in (0, 1):
            s = cc + slot
            store(slot, s).wait()
            fire(slot, s)
            wait_g(1 - slot)
            store(1 - slot, s - 1).start()

    cur_g, g_slot = s_even - 1, 1
    cur_st, st_slot = s_even - 2, 0
    if s_count % 2 == 1:
        s = s_count - 1
        store(st_slot, cur_st).wait()
        fire(st_slot, s)
        wait_g(g_slot)
        store(g_slot, cur_g).start()
        cur_g, g_slot = s, st_slot
        cur_st, st_slot = s - 1, 1 - st_slot
    if tail:
        tslot = 1 - g_slot
        store(tslot, cur_st).wait()
        fire_tail(tslot)
        wait_g(g_slot)
        store(g_slot, cur_g).start()
        for c in tail_gathers(tslot):
            c.wait()
        tail_store(tslot).start()
        store(g_slot, cur_g).wait()
        tail_store(tslot).wait()
    else:
        wait_g(g_slot)
        store(g_slot, cur_g).start()
        store(st_slot, cur_st).wait()
        store(g_slot, cur_g).wait()


def _sc_gather_full(table, idx):
    b = idx.shape[0]
    n_per = b // _NW
    s_count = n_per // _SCH
    assert n_per % _SCH == 0 and b % _NW == 0

    @functools.partial(
        pl.kernel,
        out_type=jax.ShapeDtypeStruct((b, 128), jnp.float32),
        mesh=_sc_mesh(),
        scratch_types=[
            pltpu.VMEM((2, _KCH, 128), jnp.int32),
            pltpu.VMEM((2, _SCH, 128), jnp.float32),
            pltpu.SemaphoreType.DMA((2,)),
            pltpu.SemaphoreType.DMA((2,)),
        ],
        compiler_params=pltpu.CompilerParams(use_tc_tiling_on_sc=False))
    def k(table_hbm, idx_hbm, out_hbm, idx_v, rows_v, gsem, ssem):
        wid = lax.axis_index("s") * _NC + lax.axis_index("c")
        base = wid * n_per
        _emit_gather_phase(table_hbm, idx_hbm, out_hbm, idx_v, rows_v,
                           gsem, ssem, base, s_count, 0, 0, 0, 128)

    return k(table, idx)


def _sc_gather_edges(o4, src, dest, tail):
    b = src.shape[0]
    n_per = b // _NW
    s_count = (n_per - sum(tail)) // _SCH
    assert n_per == s_count * _SCH + sum(tail) and s_count >= 2

    @functools.partial(
        pl.kernel,
        out_type=jax.ShapeDtypeStruct((b, 128), jnp.float32),
        mesh=_sc_mesh(),
        scratch_types=[
            pltpu.VMEM((2, _KCH, 128), jnp.int32),
            pltpu.VMEM((2, _SCH, 128), jnp.float32),
            pltpu.SemaphoreType.DMA((2,)),
            pltpu.SemaphoreType.DMA((2,)),
        ],
        compiler_params=pltpu.CompilerParams(use_tc_tiling_on_sc=False))
    def k(o4_hbm, src_hbm, dest_hbm, out_hbm, idx_v, rows_v, gsem, ssem):
        wid = lax.axis_index("s") * _NC + lax.axis_index("c")
        base = wid * n_per
        _emit_gather_phase(o4_hbm, src_hbm, out_hbm, idx_v, rows_v,
                           gsem, ssem, base, s_count, tail, 0, 0, 64)
        _emit_gather_phase(o4_hbm, dest_hbm, out_hbm, idx_v, rows_v,
                           gsem, ssem, base, s_count, tail, 64, 64, 64)

    return k(o4, src, dest)


def _sc_segsum(upd2, idx_ev, idx_od, n_out, tail=0):
    m = upd2.shape[0]
    rows_per = m // _NS
    nch = (rows_per - tail) // 64
    assert rows_per == nch * 64 + tail and nch % 2 == 0 and m % _NS == 0
    acc_rows = n_out + 48
    junk = n_out
    zrows = n_out // _NS

    @functools.partial(
        pl.kernel,
        out_type=jax.ShapeDtypeStruct((n_out, 128), jnp.float32),
        mesh=_sc_mesh(),
        scratch_types=[
            pltpu.VMEM((2, 2, 64), jnp.int32),
            pltpu.VMEM((2, 2, 64, 32), jnp.float32),
            pltpu.VMEM((128, 32), jnp.float32),
            pltpu.VMEM_SHARED((acc_rows, 32), jnp.float32),
            pltpu.SemaphoreType.DMA((2,)),
        ],
        compiler_params=pltpu.CompilerParams(use_tc_tiling_on_sc=False))
    def k(upd_hbm, ev_hbm, od_hbm, out_hbm, idx_v, upd_v, zbuf, acc, lsem):
        cid = lax.axis_index("c")
        sid = lax.axis_index("s")
        col0 = cid * 32
        base = sid * rows_per

        def dmas(slot, r0):
            return [
                pltpu.make_async_copy(ev_hbm.at[pl.ds(r0, 64)],
                                      idx_v.at[0, slot], lsem.at[slot]),
                pltpu.make_async_copy(od_hbm.at[pl.ds(r0, 64)],
                                      idx_v.at[1, slot], lsem.at[slot]),
                pltpu.make_async_copy(
                    upd_hbm.at[pl.ds(r0, 64), pl.ds(col0, 32)],
                    upd_v.at[0, slot], lsem.at[slot]),
                pltpu.make_async_copy(
                    upd_hbm.at[pl.ds(r0, 64), pl.ds(col0 + 64, 32)],
                    upd_v.at[1, slot], lsem.at[slot]),
            ]

        def fire(slot, r0):
            for c in dmas(slot, r0):
                c.start()

        def wait(slot, r0):
            for c in dmas(slot, r0):
                c.wait()

        def scatter(slot):
            pltpu.sync_copy(upd_v.at[0, slot], acc.at[idx_v.at[0, slot]],
                            add=True)
            pltpu.sync_copy(upd_v.at[1, slot], acc.at[idx_v.at[1, slot]],
                            add=True)

        @pl.loop(0, 128)
        def _(i):
            zbuf[i, pl.ds(0, 16)] = jnp.zeros((16,), jnp.float32)
            zbuf[i, pl.ds(16, 16)] = jnp.zeros((16,), jnp.float32)
        nz_full, z_rem = divmod(zrows, 128)
        for j in range(nz_full):
            pltpu.sync_copy(zbuf, acc.at[pl.ds(sid * zrows + j * 128, 128)])
        if z_rem:
            pltpu.sync_copy(zbuf.at[pl.ds(0, z_rem)],
                            acc.at[pl.ds(sid * zrows + nz_full * 128, z_rem)])
        plsc.subcore_barrier()

        fire(0, base)
        fire(1, base + 64)

        @pl.loop(0, nch - 2, step=2)
        def _(cc):
            for slot in (0, 1):
                r0 = base + (cc + slot) * 64
                wait(slot, r0)
                scatter(slot)
                fire(slot, r0 + 128)

        for slot in (0, 1):
            r0 = base + (nch - 2 + slot) * 64
            wait(slot, r0)
            scatter(slot)

        if tail:
            r0 = base + nch * 64
            for kk in range(0, 64, 16):
                idx_v[0, 0, pl.ds(kk, 16)] = jnp.full((16,), junk, jnp.int32)
                idx_v[1, 0, pl.ds(kk, 16)] = jnp.full((16,), junk, jnp.int32)
            pltpu.sync_copy(ev_hbm.at[pl.ds(r0, tail)],
                            idx_v.at[0, 0, pl.ds(0, tail)])
            pltpu.sync_copy(od_hbm.at[pl.ds(r0, tail)],
                            idx_v.at[1, 0, pl.ds(0, tail)])
            pltpu.sync_copy(upd_hbm.at[pl.ds(r0, tail), pl.ds(col0, 32)],
                            upd_v.at[0, 0, pl.ds(0, tail)])
            pltpu.sync_copy(upd_hbm.at[pl.ds(r0, tail), pl.ds(col0 + 64, 32)],
                            upd_v.at[1, 0, pl.ds(0, tail)])
            scatter(0)

        plsc.subcore_barrier()

        pltpu.sync_copy(acc.at[pl.ds(sid * zrows, zrows)],
                        out_hbm.at[pl.ds(sid * zrows, zrows), pl.ds(col0, 32)])

    return k(upd2, idx_ev, idx_od)


def kernel(bus, shunt, gen, load, branch_attr, shunt_to_bus, gen_to_bus,
           load_to_bus, branch_index, p_bus, p_shunt, p_gen, p_load, p_branch,
           p_bb, p_bn):
    n_bus = bus.shape[0]
    n_shunt = shunt.shape[0]
    n_gen = gen.shape[0]

    n_items = n_shunt + n_gen + load.shape[0]
    b_items = _NW * _SCH * 5
    idx_items = jnp.concatenate([
        shunt_to_bus, gen_to_bus, load_to_bus,
        jnp.arange(b_items - n_items, dtype=jnp.int32) % n_bus])
    g_bus = _sc_gather_full(bus, idx_items)

    w1b = p_bus['W1']
    shunt_o, u_s = _item_mlp(shunt, g_bus, 0, p_shunt['W1'][:16],
                             p_shunt['W1'][16:], p_shunt['b1'][None],
                             p_shunt['W2'], p_shunt['b2'][None], w1b[128:192])
    gen_o, u_g = _item_mlp(gen, g_bus, n_shunt, p_gen['W1'][:16],
                           p_gen['W1'][16:], p_gen['b1'][None],
                           p_gen['W2'], p_gen['b2'][None], w1b[192:256])
    load_o, u_l = _item_mlp(load, g_bus, n_shunt + n_gen, p_load['W1'][:16],
                            p_load['W1'][16:], p_load['b1'][None],
                            p_load['W2'], p_load['b2'][None], w1b[256:320])

    m_items = n_items // 2
    m_pad = 36864
    upd2 = jnp.concatenate(
        [u_s, u_g, u_l,
         jnp.zeros((m_pad - m_items, 128), jnp.float32)], axis=0)
    idx = jnp.concatenate([shunt_to_bus, gen_to_bus, load_to_bus])
    idx2 = idx.reshape(-1, 2, 1000)
    pad_i = jnp.full((m_pad - m_items,), n_bus, jnp.int32)
    idx_ev = jnp.concatenate([idx2[:, 0].ravel(), pad_i])
    idx_od = jnp.concatenate([idx2[:, 1].ravel(), pad_i])
    agg = _sc_segsum(upd2, idx_ev, idx_od, n_bus)

    o4 = _bus_mlp(bus, agg, w1b[:128], p_bus['b1'][None], p_bus['W2'],
                  p_bus['b2'][None], p_branch['W1'][:64], p_branch['b1'][None])

    src = branch_index[0]
    dest = branch_index[1]
    w1br = p_branch['W1']
    w1n = p_bb['W1']
    attr_t = branch_attr.T
    h1 = 409600
    halves = ((0, h1, (128, 128, 64), 0),
              (h1, src.shape[0] - h1, (128, 72), 40))
    bo_ts, bns = [], []
    for e0, nh, gtail, stail in halves:
        g_edge = _sc_gather_edges(o4, src[e0:e0 + nh], dest[e0:e0 + nh],
                                  gtail)
        bo_t, bn_e2 = _edge_mlp(
            g_edge, attr_t, e0, w1br[64:80], w1br[80:144], p_branch['W2'],
            p_branch['b2'][:, None], w1n[64:80], w1n[:64], p_bb['b1'][None],
            p_bb['W2'], p_bb['b2'][None])
        bo_ts.append(bo_t)
        srch = src[e0:e0 + nh].reshape(-1, 2, 1600)
        bns.append(_sc_segsum(bn_e2, srch[:, 0].ravel(), srch[:, 1].ravel(),
                              n_bus, tail=stail))
    branch_o = jnp.concatenate(bo_ts, axis=1).T

    w1f = p_bn['W1']
    bus_out = _final_mlp(bns[0], bns[1], o4, w1f[:64], w1f[64:128],
                         p_bn['b1'][None], p_bn['W2'], p_bn['b2'][None])

    return (bus_out, shunt_o, gen_o, load_o, branch_o)


# speedup vs baseline: 1.6179x; 1.0311x over previous
import functools

import jax
import jax.numpy as jnp
from jax import lax
from jax.experimental import pallas as pl
from jax.experimental.pallas import tpu as pltpu
from jax.experimental.pallas import tpu_sc as plsc


def _rep(shape):
    return pl.BlockSpec(shape, lambda *_: (0,) * len(shape))


def _row(bm, cols):
    return pl.BlockSpec((bm, cols), lambda i: (i, 0))


def _dot(a, b):
    return jnp.dot(a, b, preferred_element_type=jnp.float32)


def _bdot(a, b):
    return jnp.dot(a.astype(jnp.bfloat16), b.astype(jnp.bfloat16),
                   preferred_element_type=jnp.float32)


def _dot_t(a_t, b):
    return lax.dot_general(a_t.astype(jnp.bfloat16), b.astype(jnp.bfloat16),
                           (((0,), (0,)), ((), ())),
                           preferred_element_type=jnp.float32)


def _dot_ot(w, h):
    return lax.dot_general(w.astype(jnp.bfloat16), h.astype(jnp.bfloat16),
                           (((0,), (1,)), ((), ())),
                           preferred_element_type=jnp.float32)


def _k2_body(x_ref, g_ref, w1_ref, wb1_ref, b1_ref, w2_ref, b2_ref, wb_ref,
             o_ref, u_ref):
    h = jax.nn.relu(_bdot(x_ref[...], w1_ref[...])
                    + _bdot(g_ref[...], wb1_ref[...]) + b1_ref[...])
    o = jax.nn.relu(_bdot(h, w2_ref[...]) + b2_ref[...])
    o_ref[...] = o
    u = _bdot(o, wb_ref[...])
    half = u.shape[0] // 2
    u_ref[...] = jnp.concatenate([u[:half], u[half:]], axis=1)


def _item_mlp(x, g, g_off, w1, wb1, b1, w2, b2, wb, bm=2000):
    n = x.shape[0]
    ob = g_off // bm
    return pl.pallas_call(
        _k2_body,
        grid=(n // bm,),
        in_specs=[_row(bm, 16),
                  pl.BlockSpec((bm, 128), lambda i: (i + ob, 0)),
                  _rep((16, 64)), _rep((128, 64)), _rep((1, 64)),
                  _rep((64, 64)), _rep((1, 64)), _rep((64, 64))],
        out_specs=[_row(bm, 64), _row(bm // 2, 128)],
        out_shape=[jax.ShapeDtypeStruct((n, 64), jnp.float32),
                   jax.ShapeDtypeStruct((n // 2, 128), jnp.float32)],
        compiler_params=pltpu.CompilerParams(dimension_semantics=("parallel",)),
    )(x, g, w1, wb1, b1, w2, b2, wb)


def _k4_body(bus_ref, agg_ref, w1_ref, b1_ref, w2_ref, b2_ref, ws_ref, bs_ref,
             o_ref):
    pre = (_dot(bus_ref[...], w1_ref[...]) + agg_ref[:, :64] + b1_ref[...])
    h = jax.nn.relu(pre)
    bus_h = jax.nn.relu(_dot(h, w2_ref[...]) + b2_ref[...])
    s = _dot(bus_h, ws_ref[...]) + bs_ref[...]
    o_ref[...] = jnp.concatenate([s, bus_h], axis=1)


def _bus_mlp(bus, agg, w1, b1, w2, b2, ws, bs, bm=2000):
    n = bus.shape[0]
    return pl.pallas_call(
        _k4_body,
        grid=(n // bm,),
        in_specs=[_row(bm, 128), _row(bm, 128), _rep((128, 64)), _rep((1, 64)),
                  _rep((64, 64)), _rep((1, 64)), _rep((64, 64)), _rep((1, 64))],
        out_specs=_row(bm, 128),
        out_shape=jax.ShapeDtypeStruct((n, 128), jnp.float32),
        compiler_params=pltpu.CompilerParams(dimension_semantics=("parallel",)),
    )(bus, agg, w1, b1, w2, b2, ws, bs)


def _k5_body(g_ref, attrt_ref, wba_ref, wbd_ref, w2b_ref, b2bt_ref,
             wna_ref, wnd_ref, b1n_ref, w2n_ref, b2n_ref, bot_ref, bne_ref):
    attrt = attrt_ref[...]
    gs = g_ref[:, :64]
    gd = g_ref[:, 64:]
    h_b = jax.nn.relu(gs + _dot_t(attrt, wba_ref[...])
                      + _bdot(gd, wbd_ref[...]))
    bot_ref[...] = jax.nn.relu(_dot_ot(w2b_ref[...], h_b) + b2bt_ref[...])
    h_n = jax.nn.relu(_bdot(gd, wnd_ref[...]) + _dot_t(attrt, wna_ref[...])
                      + b1n_ref[...])
    bne = jax.nn.relu(_bdot(h_n, w2n_ref[...]) + b2n_ref[...])
    half = bne.shape[0] // 2
    bne_ref[...] = jnp.concatenate([bne[:half], bne[half:]], axis=1)


def _edge_mlp(g, attr_t, e_off, wba, wbd, w2b, b2bt, wna, wnd, b1n, w2n, b2n,
              bm=6400):
    n = g.shape[0]
    ob = e_off // bm
    bo_t, bne = pl.pallas_call(
        _k5_body,
        grid=(n // bm,),
        in_specs=[_row(bm, 128),
                  pl.BlockSpec((16, bm), lambda i: (0, i + ob)),
                  _rep((16, 64)),
                  _rep((64, 64)), _rep((64, 16)), _rep((16, 1)), _rep((16, 64)),
                  _rep((64, 64)), _rep((1, 64)), _rep((64, 64)), _rep((1, 64))],
        out_specs=[pl.BlockSpec((16, bm), lambda i: (0, i)),
                   _row(bm // 2, 128)],
        out_shape=[jax.ShapeDtypeStruct((16, n), jnp.float32),
                   jax.ShapeDtypeStruct((n // 2, 128), jnp.float32)],
        compiler_params=pltpu.CompilerParams(dimension_semantics=("parallel",)),
    )(g, attr_t, wba, wbd, w2b, b2bt, wna, wnd, b1n, w2n, b2n)
    return bo_t, bne


def _k6_body(bn1_ref, bn2_ref, o4_ref, wa_ref, wb_ref, b1_ref, w2_ref, b2_ref,
             o_ref):
    bn = bn1_ref[:, :64] + bn2_ref[:, :64]
    pre = (_bdot(bn, wa_ref[...]) + _bdot(o4_ref[:, 64:], wb_ref[...])
           + b1_ref[...])
    h = jax.nn.relu(pre)
    o_ref[...] = jax.nn.relu(_bdot(h, w2_ref[...]) + b2_ref[...])


def _final_mlp(bn1, bn2, o4, wa, wb, b1, w2, b2, bm=2000):
    n = bn1.shape[0]
    return pl.pallas_call(
        _k6_body,
        grid=(n // bm,),
        in_specs=[_row(bm, 128), _row(bm, 128), _row(bm, 128),
                  _rep((64, 64)), _rep((64, 64)),
                  _rep((1, 64)), _rep((64, 128)), _rep((1, 128))],
        out_specs=_row(bm, 128),
        out_shape=jax.ShapeDtypeStruct((n, 128), jnp.float32),
        compiler_params=pltpu.CompilerParams(dimension_semantics=("parallel",)),
    )(bn1, bn2, o4, wa, wb, b1, w2, b2)


_NC = 2
_NS = 16
_NW = _NC * _NS
_CH = 120
_KCH = 4
_SCH = _CH * _KCH


def _sc_mesh():
    return plsc.VectorSubcoreMesh(core_axis_name="c", subcore_axis_name="s",
                                  num_cores=_NC, num_subcores=_NS)


def _emit_gather_phase(table_hbm, idx_hbm, out_hbm, idx_v, rows_v, gsem, ssem,
                       base, s_count, tail, src_col, dst_col, width):
    def load_idx(slot, s):
        for j in range(_KCH):
            pltpu.sync_copy(
                idx_hbm.at[pl.ds(base + s * _SCH + j * _CH, _CH)],
                idx_v.at[slot, j, pl.ds(0, _CH)])

    def gathers(slot):
        return [pltpu.make_async_copy(
            table_hbm.at[idx_v.at[slot, j, pl.ds(0, _CH)]],
            rows_v.at[slot, pl.ds(j * _CH, _CH)],
            gsem.at[slot]) for j in range(_KCH)]

    def fire(slot, s):
        load_idx(slot, s)
        for c in gathers(slot):
            c.start()

    def wait_g(slot):
        for c in gathers(slot):
            c.wait()

    def store(slot, s, nrows=_SCH):
        return pltpu.make_async_copy(
            rows_v.at[slot, pl.ds(0, nrows), pl.ds(src_col, width)],
            out_hbm.at[pl.ds(base + s * _SCH, nrows), pl.ds(dst_col, width)],
            ssem.at[slot])

    def tail_gathers(slot):
        cs = []
        off = 0
        for j, tc in enumerate(tail):
            cs.append(pltpu.make_async_copy(
                table_hbm.at[idx_v.at[slot, j, pl.ds(0, tc)]],
                rows_v.at[slot, pl.ds(off, tc)], gsem.at[slot]))
            off += tc
        return cs

    def fire_tail(slot):
        t0 = base + s_count * _SCH
        off = 0
        for j, tc in enumerate(tail):
            pltpu.sync_copy(idx_hbm.at[pl.ds(t0 + off, tc)],
                            idx_v.at[slot, j, pl.ds(0, tc)])
            off += tc
        for c in tail_gathers(slot):
            c.start()

    def tail_store(slot):
        t0 = base + s_count * _SCH
        trows = sum(tail)
        return pltpu.make_async_copy(
            rows_v.at[slot, pl.ds(0, trows), pl.ds(src_col, width)],
            out_hbm.at[pl.ds(t0, trows), pl.ds(dst_col, width)],
            ssem.at[slot])

    fire(0, 0)
    fire(1, 1)
    wait_g(0)
    store(0, 0).start()

    s_even = s_count - (s_count % 2)

    @pl.loop(2, s_even, step=2)
    def _(cc):
        for slot in (0, 1):
            s = cc + slot
            store(slot, s).wait()
            fire(slot, s)
            wait_g(1 - slot)
            store(1 - slot, s - 1).start()

    cur_g, g_slot = s_even - 1, 1
    cur_st, st_slot = s_even - 2, 0
    if s_count % 2 == 1:
        s = s_count - 1
        store(st_slot, cur_st).wait()
        fire(st_slot, s)
        wait_g(g_slot)
        store(g_slot, cur_g).start()
        cur_g, g_slot = s, st_slot
        cur_st, st_slot = s - 1, 1 - st_slot
    if tail:
        tslot = 1 - g_slot
        store(tslot, cur_st).wait()
        fire_tail(tslot)
        wait_g(g_slot)
        store(g_slot, cur_g).start()
        for c in tail_gathers(tslot):
            c.wait()
        tail_store(tslot).start()
        store(g_slot, cur_g).wait()
        tail_store(tslot).wait()
    else:
        wait_g(g_slot)
        store(g_slot, cur_g).start()
        store(st_slot, cur_st).wait()
        store(g_slot, cur_g).wait()


def _sc_gather_full(table, idx):
    b = idx.shape[0]
    n_per = b // _NW
    s_count = n_per // _SCH
    assert n_per % _SCH == 0 and b % _NW == 0

    @functools.partial(
        pl.kernel,
        out_type=jax.ShapeDtypeStruct((b, 128), jnp.float32),
        mesh=_sc_mesh(),
        scratch_types=[
            pltpu.VMEM((2, _KCH, 128), jnp.int32),
            pltpu.VMEM((2, _SCH, 128), jnp.float32),
            pltpu.SemaphoreType.DMA((2,)),
            pltpu.SemaphoreType.DMA((2,)),
        ],
        compiler_params=pltpu.CompilerParams(use_tc_tiling_on_sc=False))
    def k(table_hbm, idx_hbm, out_hbm, idx_v, rows_v, gsem, ssem):
        wid = lax.axis_index("s") * _NC + lax.axis_index("c")
        base = wid * n_per
        _emit_gather_phase(table_hbm, idx_hbm, out_hbm, idx_v, rows_v,
                           gsem, ssem, base, s_count, 0, 0, 0, 128)

    return k(table, idx)


def _sc_gather_edges(o4, src, dest, tail):
    b = src.shape[0]
    n_per = b // _NW
    s_count = (n_per - sum(tail)) // _SCH
    assert n_per == s_count * _SCH + sum(tail) and s_count >= 2

    @functools.partial(
        pl.kernel,
        out_type=jax.ShapeDtypeStruct((b, 128), jnp.float32),
        mesh=_sc_mesh(),
        scratch_types=[
            pltpu.VMEM((2, _KCH, 128), jnp.int32),
            pltpu.VMEM((2, _SCH, 128), jnp.float32),
            pltpu.SemaphoreType.DMA((2,)),
            pltpu.SemaphoreType.DMA((2,)),
        ],
        compiler_params=pltpu.CompilerParams(use_tc_tiling_on_sc=False))
    def k(o4_hbm, src_hbm, dest_hbm, out_hbm, idx_v, rows_v, gsem, ssem):
        wid = lax.axis_index("s") * _NC + lax.axis_index("c")
        base = wid * n_per
        _emit_gather_phase(o4_hbm, src_hbm, out_hbm, idx_v, rows_v,
                           gsem, ssem, base, s_count, tail, 0, 0, 64)
        _emit_gather_phase(o4_hbm, dest_hbm, out_hbm, idx_v, rows_v,
                           gsem, ssem, base, s_count, tail, 64, 64, 64)

    return k(o4, src, dest)


def _sc_segsum(upd2, idx_ev, idx_od, n_out, tail=0):
    m = upd2.shape[0]
    rows_per = m // _NS
    nch = (rows_per - tail) // 64
    assert rows_per == nch * 64 + tail and nch % 2 == 0 and m % _NS == 0
    acc_rows = n_out + 48
    junk = n_out
    zrows = n_out // _NS

    @functools.partial(
        pl.kernel,
        out_type=jax.ShapeDtypeStruct((n_out, 128), jnp.float32),
        mesh=_sc_mesh(),
        scratch_types=[
            pltpu.VMEM((2, 2, 64), jnp.int32),
            pltpu.VMEM((2, 2, 64, 32), jnp.float32),
            pltpu.VMEM((128, 32), jnp.float32),
            pltpu.VMEM_SHARED((acc_rows, 32), jnp.float32),
            pltpu.SemaphoreType.DMA((2,)),
        ],
        compiler_params=pltpu.CompilerParams(use_tc_tiling_on_sc=False))
    def k(upd_hbm, ev_hbm, od_hbm, out_hbm, idx_v, upd_v, zbuf, acc, lsem):
        cid = lax.axis_index("c")
        sid = lax.axis_index("s")
        col0 = cid * 32
        base = sid * rows_per

        def dmas(slot, r0):
            return [
                pltpu.make_async_copy(ev_hbm.at[pl.ds(r0, 64)],
                                      idx_v.at[0, slot], lsem.at[slot]),
                pltpu.make_async_copy(od_hbm.at[pl.ds(r0, 64)],
                                      idx_v.at[1, slot], lsem.at[slot]),
                pltpu.make_async_copy(
                    upd_hbm.at[pl.ds(r0, 64), pl.ds(col0, 32)],
                    upd_v.at[0, slot], lsem.at[slot]),
                pltpu.make_async_copy(
                    upd_hbm.at[pl.ds(r0, 64), pl.ds(col0 + 64, 32)],
                    upd_v.at[1, slot], lsem.at[slot]),
            ]

        def fire(slot, r0):
            for c in dmas(slot, r0):
                c.start()

        def wait(slot, r0):
            for c in dmas(slot, r0):
                c.wait()

        def scatter(slot):
            pltpu.sync_copy(upd_v.at[0, slot], acc.at[idx_v.at[0, slot]],
                            add=True)
            pltpu.sync_copy(upd_v.at[1, slot], acc.at[idx_v.at[1, slot]],
                            add=True)

        @pl.loop(0, 128)
        def _(i):
            zbuf[i, pl.ds(0, 16)] = jnp.zeros((16,), jnp.float32)
            zbuf[i, pl.ds(16, 16)] = jnp.zeros((16,), jnp.float32)
        nz_full, z_rem = divmod(zrows, 128)
        for j in range(nz_full):
            pltpu.sync_copy(zbuf, acc.at[pl.ds(sid * zrows + j * 128, 128)])
        if z_rem:
            pltpu.sync_copy(zbuf.at[pl.ds(0, z_rem)],
                            acc.at[pl.ds(sid * zrows + nz_full * 128, z_rem)])
        plsc.subcore_barrier()

        fire(0, base)
        fire(1, base + 64)

        @pl.loop(0, nch - 2, step=2)
        def _(cc):
            for slot in (0, 1):
                r0 = base + (cc + slot) * 64
                wait(slot, r0)
                scatter(slot)
                fire(slot, r0 + 128)

        for slot in (0, 1):
            r0 = base + (nch - 2 + slot) * 64
            wait(slot, r0)
            scatter(slot)

        if tail:
            r0 = base + nch * 64
            for kk in range(0, 64, 16):
                idx_v[0, 0, pl.ds(kk, 16)] = jnp.full((16,), junk, jnp.int32)
                idx_v[1, 0, pl.ds(kk, 16)] = jnp.full((16,), junk, jnp.int32)
            pltpu.sync_copy(ev_hbm.at[pl.ds(r0, tail)],
                            idx_v.at[0, 0, pl.ds(0, tail)])
            pltpu.sync_copy(od_hbm.at[pl.ds(r0, tail)],
                            idx_v.at[1, 0, pl.ds(0, tail)])
            pltpu.sync_copy(upd_hbm.at[pl.ds(r0, tail), pl.ds(col0, 32)],
                            upd_v.at[0, 0, pl.ds(0, tail)])
            pltpu.sync_copy(upd_hbm.at[pl.ds(r0, tail), pl.ds(col0 + 64, 32)],
                            upd_v.at[1, 0, pl.ds(0, tail)])
            scatter(0)

        plsc.subcore_barrier()

        pltpu.sync_copy(acc.at[pl.ds(sid * zrows, zrows)],
                        out_hbm.at[pl.ds(sid * zrows, zrows), pl.ds(col0, 32)])

    return k(upd2, idx_ev, idx_od)


def kernel(bus, shunt, gen, load, branch_attr, shunt_to_bus, gen_to_bus,
           load_to_bus, branch_index, p_bus, p_shunt, p_gen, p_load, p_branch,
           p_bb, p_bn):
    n_bus = bus.shape[0]
    n_shunt = shunt.shape[0]
    n_gen = gen.shape[0]

    n_items = n_shunt + n_gen + load.shape[0]
    b_items = _NW * _SCH * 5
    idx_items = jnp.concatenate([
        shunt_to_bus, gen_to_bus, load_to_bus,
        jnp.arange(b_items - n_items, dtype=jnp.int32) % n_bus])
    g_bus = _sc_gather_full(bus, idx_items)

    w1b = p_bus['W1']
    shunt_o, u_s = _item_mlp(shunt, g_bus, 0, p_shunt['W1'][:16],
                             p_shunt['W1'][16:], p_shunt['b1'][None],
                             p_shunt['W2'], p_shunt['b2'][None], w1b[128:192])
    gen_o, u_g = _item_mlp(gen, g_bus, n_shunt, p_gen['W1'][:16],
                           p_gen['W1'][16:], p_gen['b1'][None],
                           p_gen['W2'], p_gen['b2'][None], w1b[192:256])
    load_o, u_l = _item_mlp(load, g_bus, n_shunt + n_gen, p_load['W1'][:16],
                            p_load['W1'][16:], p_load['b1'][None],
                            p_load['W2'], p_load['b2'][None], w1b[256:320])

    m_items = n_items // 2
    m_pad = 36864
    upd2 = jnp.concatenate(
        [u_s, u_g, u_l,
         jnp.zeros((m_pad - m_items, 128), jnp.float32)], axis=0)
    idx = jnp.concatenate([shunt_to_bus, gen_to_bus, load_to_bus])
    idx2 = idx.reshape(-1, 2, 1000)
    pad_i = jnp.full((m_pad - m_items,), n_bus, jnp.int32)
    idx_ev = jnp.concatenate([idx2[:, 0].ravel(), pad_i])
    idx_od = jnp.concatenate([idx2[:, 1].ravel(), pad_i])
    agg = _sc_segsum(upd2, idx_ev, idx_od, n_bus)

    o4 = _bus_mlp(bus, agg, w1b[:128], p_bus['b1'][None], p_bus['W2'],
                  p_bus['b2'][None], p_branch['W1'][:64], p_branch['b1'][None])

    src = branch_index[0]
    dest = branch_index[1]
    w1br = p_branch['W1']
    w1n = p_bb['W1']
    attr_t = branch_attr.T
    h1 = 409600
    halves = ((0, h1, (128, 128, 64), 0),
              (h1, src.shape[0] - h1, (128, 72), 40))
    bo_ts, bns = [], []
    for e0, nh, gtail, stail in halves:
        g_edge = _sc_gather_edges(o4, src[e0:e0 + nh], dest[e0:e0 + nh],
                                  gtail)
        bo_t, bn_e2 = _edge_mlp(
            g_edge, attr_t, e0, w1br[64:80], w1br[80:144], p_branch['W2'],
            p_branch['b2'][:, None], w1n[64:80], w1n[:64], p_bb['b1'][None],
            p_bb['W2'], p_bb['b2'][None])
        bo_ts.append(bo_t)
        srch = src[e0:e0 + nh].reshape(-1, 2, 3200)
        bns.append(_sc_segsum(bn_e2, srch[:, 0].ravel(), srch[:, 1].ravel(),
                              n_bus, tail=stail))
    branch_o = jnp.concatenate(bo_ts, axis=1).T

    w1f = p_bn['W1']
    bus_out = _final_mlp(bns[0], bns[1], o4, w1f[:64], w1f[64:128],
                         p_bn['b1'][None], p_bn['W2'], p_bn['b2'][None])

    return (bus_out, shunt_o, gen_o, load_o, branch_o)
